# baseline (speedup 1.0000x reference)
import functools

import jax
import jax.numpy as jnp
from jax import lax
from jax.experimental import pallas as pl
from jax.experimental.pallas import tpu as pltpu
from jax.experimental.pallas import tpu_sc as plsc

B = 16384
NE = 1000000
D = 32
L = 16
NW = 32

CHUNKS = 7813
TAIL_CHUNK = CHUNKS - 1
TAIL_BASE = TAIL_CHUNK * 128
CH_PER = 245
PH_CH = 4
PH_E = PH_CH * 128
NPH = 62
QTOT = 2 * B
GROWS = QTOT + L
QPW = B // NW
QPIECE = 2048


def _rsqrt(x):
    i = plsc.bitcast(x, jnp.int32)
    i = jnp.int32(0x5F3759DF) - (i >> 1)
    y = plsc.bitcast(i, jnp.float32)
    for _ in range(3):
        y = y * (1.5 - 0.5 * x * y * y)
    return y


def _gather_body(entT_hbm, tail_hbm, e1_hbm, e2_hbm, g_hbm,
                 epiece, mpk, mpk2, qb2, rowstage, slab3,
                 starts_s, curs_s, dsem0, dsem1,
                 ssem0, ssem1, ssem2, ssem3, ssem4, ssem5, ssem6, ssem7):
    c = lax.axis_index("c")
    s = lax.axis_index("s")
    wid = s * 2 + c
    lo_ch = wid * CH_PER
    hi_ch = jnp.minimum(lo_ch + CH_PER, CHUNKS)
    base_e = lo_ch * 128
    span_e = jnp.minimum(hi_ch * 128, NE) - base_e

    iota = lax.iota(jnp.int32, L)

    def issue(ph, bsel, sem):
        for k in range(PH_CH):
            cg = lo_ch + ph * PH_CH + k
            valid = cg < hi_ch
            is_tail = cg == TAIL_CHUNK
            src_off = pl.multiple_of(
                jnp.where(valid & (~is_tail), cg, 0) * 128, 128)
            dst = slab3.at[bsel, :, pl.ds(k * 128, 128)]

            @pl.when(valid & (~is_tail))
            def _():
                pltpu.async_copy(
                    entT_hbm.at[:, pl.ds(src_off, 128)], dst, sem)

            @pl.when(valid & is_tail)
            def _():
                pltpu.async_copy(tail_hbm, dst, sem)

            @pl.when(~valid)
            def _():
                pltpu.async_copy(
                    entT_hbm.at[:, pl.ds(0, 128)], dst, sem)

    issue(0, 0, dsem0)
    issue(1, 1, dsem1)

    m_total = 0
    for pi in range(QTOT // QPIECE):
        src = e1_hbm if pi < (B // QPIECE) else e2_hbm
        off = pi * QPIECE if pi < (B // QPIECE) else (pi * QPIECE - B)
        pltpu.sync_copy(src.at[pl.ds(off, QPIECE)], epiece)

        def match(i, n, pi=pi):
            evec = epiece[pl.ds(i * L, L)]
            qvec = pi * QPIECE + i * L + iota
            eloc = evec - base_e
            mask = (eloc >= 0) & (eloc < span_e)
            packed = eloc * 32768 + qvec
            plsc.store_compressed(mpk.at[pl.ds(n, L)], packed, mask=mask)
            return n + plsc.all_reduce_population_count(mask)[0]

        m_total = lax.fori_loop(0, QPIECE // L, match, m_total)

    for ph in range(NPH + 1):
        curs_s[ph] = 0

    def hist(i, _):
        pk = mpk[pl.ds(i * L, L)]
        valid = (i * L + iota) < m_total
        pvec = jnp.where(valid, pk >> 24, NPH)
        for j in range(L):
            pj = pvec[j]
            curs_s[pj] = curs_s[pj] + 1
        return 0

    lax.fori_loop(0, (m_total + L - 1) // L, hist, 0)

    starts_s[0] = 0
    for ph in range(NPH + 1):
        starts_s[ph + 1] = starts_s[ph] + curs_s[ph]
    for ph in range(NPH + 1):
        curs_s[ph] = starts_s[ph]

    lane0 = iota == 0
    zi = jnp.zeros((L,), jnp.int32)

    def place(i, _):
        pk = mpk[pl.ds(i * L, L)]
        valid = (i * L + iota) < m_total
        pvec = jnp.where(valid, pk >> 24, NPH)
        for j in range(L):
            pj = pvec[j]
            pos = curs_s[pj]
            curs_s[pj] = pos + 1
            plsc.store_scatter(mpk2, [zi + pos], zi + pk[j], mask=lane0)
        return 0

    lax.fori_loop(0, (m_total + L - 1) // L, place, 0)

    ssems = [ssem0, ssem1, ssem2, ssem3, ssem4, ssem5, ssem6, ssem7]
    dsems = [dsem0, dsem1]

    def _drain_scatter(sk):
        pltpu.make_async_copy(
            rowstage.at[pl.ds(0, L), :], g_hbm.at[pl.ds(0, L), :], sk).wait()

    def phase(ph, gi):
        bsel = lax.rem(ph, 2)

        for bb in range(2):
            @pl.when(bsel == bb)
            def _(bb=bb):
                pltpu.make_async_copy(
                    entT_hbm.at[:, pl.ds(0, PH_E)], slab3.at[bb],
                    dsems[bb]).wait()

        n0 = starts_s[ph]
        n1 = starts_s[ph + 1]
        slab_lo = ph * PH_E
        b16 = jnp.zeros((L,), jnp.int32) + bsel

        def group(g, gi2):
            slot = gi2 & 7
            i0 = n0 + g * L
            pk = mpk2[pl.ds(i0, L)]
            valid = (i0 + iota) < n1
            eloc = jnp.where(valid, (pk >> 15) - slab_lo, 0)
            qv = jnp.where(valid, pk & 32767, QTOT + iota)
            for sk in range(8):
                @pl.when((slot == sk) & (gi2 >= 8))
                def _(sk=sk):
                    _drain_scatter(ssems[sk])
            rows = slot * L + iota
            for d in range(D):
                dv = jnp.full((L,), d, jnp.int32)
                v = plsc.load_gather(slab3, [b16, dv, eloc])
                col = (16 * (d // 16)) + ((d + qv) & 15)
                plsc.store_scatter(rowstage, [rows, col], v)
            plsc.store_scatter(qb2, [jnp.zeros((L,), jnp.int32) + slot, iota],
                               qv)
            for sk in range(8):
                @pl.when(slot == sk)
                def _(sk=sk):
                    pltpu.async_copy(
                        rowstage.at[pl.ds(sk * L, L), :],
                        g_hbm.at[qb2.at[sk]], ssems[sk])
            return gi2 + 1

        ngroups = (n1 - n0 + L - 1) // L
        gi = lax.fori_loop(0, ngroups, group, gi)

        for bb in range(2):
            @pl.when((bsel == bb) & (ph + 2 < NPH))
            def _(bb=bb):
                issue(ph + 2, bb, dsems[bb])

        return gi

    gi = lax.fori_loop(0, NPH, phase, 0)

    for sk in range(8):
        @pl.when(sk < jnp.minimum(gi, 8))
        def _(sk=sk):
            _drain_scatter(ssems[sk])


def _compute_body(g_hbm, relt_hbm, ridx_hbm, out_hbm,
                  relv, ridx, hbuf, tbuf, outv, dsem):
    c = lax.axis_index("c")
    s = lax.axis_index("s")
    wid = s * 2 + c
    qlo = wid * QPW

    pltpu.sync_copy(relt_hbm, relv)
    pltpu.sync_copy(ridx_hbm.at[pl.ds(qlo, QPW)], ridx)

    iota = lax.iota(jnp.int32, L)
    cols = [((iota + m) & (L - 1)) for m in range(L)]
    zero = jnp.zeros((L,), jnp.float32)

    for p in range(QPW // 128):
        pltpu.sync_copy(g_hbm.at[pl.ds(qlo + p * 128, 128), :], hbuf)
        pltpu.sync_copy(g_hbm.at[pl.ds(B + qlo + p * 128, 128), :], tbuf)

        def group(g, _, p=p):
            rows = g * L + iota
            qbase = qlo + p * 128 + g * L
            qb15 = qbase & 15
            rho = ridx[pl.ds(p * 128 + g * L, L)]
            hh = tt = rr = hr = ht = rt = zero
            for m in range(L):
                for hi in (0, L):
                    colv = cols[m] + hi if hi else cols[m]
                    f = hi + ((m - qb15) & 15)
                    fv = jnp.zeros((L,), jnp.int32) + f
                    h = plsc.load_gather(hbuf, [rows, colv])
                    t = plsc.load_gather(tbuf, [rows, colv])
                    r = plsc.load_gather(relv, [fv, rho])
                    hh = hh + h * h
                    tt = tt + t * t
                    rr = rr + r * r
                    hr = hr + h * r
                    ht = ht + h * t
                    rt = rt + r * t
            ch = _rsqrt(jnp.maximum(hh, 1e-24))
            ct = _rsqrt(jnp.maximum(tt, 1e-24))
            eh = hh * ch * ch
            et = tt * ct * ct
            d2 = eh + et + rr + 2.0 * (hr * ch - ht * (ch * ct) - rt * ct)
            d2 = jnp.maximum(d2, 0.0)
            dist = d2 * _rsqrt(jnp.maximum(d2, 1e-30))
            plsc.store_scatter(outv, [p * 128 + rows], dist)
            return 0

        lax.fori_loop(0, 128 // L, group, 0)

    pltpu.sync_copy(outv, out_hbm.at[pl.ds(qlo, QPW)])


@jax.jit
def _transe(e1_idx, rel_idx, e2_idx, emb_ent, emb_rel):
    entT = emb_ent.T
    tailT = jnp.pad(emb_ent[TAIL_BASE:].T, ((0, 0), (0, 64)))
    relT = jnp.pad(emb_rel.T, ((0, 0), (0, 24)))
    mesh = plsc.VectorSubcoreMesh(core_axis_name="c", subcore_axis_name="s")
    cp = pltpu.CompilerParams(
        needs_layout_passes=False, use_tc_tiling_on_sc=True)

    gather = pl.kernel(
        _gather_body,
        out_type=jax.ShapeDtypeStruct((GROWS, 128), jnp.float32),
        mesh=mesh,
        compiler_params=cp,
        scratch_types=[
            pltpu.VMEM((QPIECE,), jnp.int32),
            pltpu.VMEM((QTOT + L,), jnp.int32),
            pltpu.VMEM((QTOT + L,), jnp.int32),
            pltpu.VMEM((8, L), jnp.int32),
            pltpu.VMEM((128, 128), jnp.float32),
            pltpu.VMEM((2, D, PH_E), jnp.float32),
            pltpu.SMEM((NPH + 2,), jnp.int32),
            pltpu.SMEM((NPH + 1,), jnp.int32),
        ] + [pltpu.SemaphoreType.DMA] * 10,
    )
    compute = pl.kernel(
        _compute_body,
        out_type=jax.ShapeDtypeStruct((B,), jnp.float32),
        mesh=mesh,
        compiler_params=cp,
        scratch_types=[
            pltpu.VMEM((D, 1024), jnp.float32),
            pltpu.VMEM((QPW,), jnp.int32),
            pltpu.VMEM((128, 128), jnp.float32),
            pltpu.VMEM((128, 128), jnp.float32),
            pltpu.VMEM((QPW,), jnp.float32),
            pltpu.SemaphoreType.DMA,
        ],
    )
    g = gather(entT, tailT, e1_idx, e2_idx)
    return compute(g, relT, rel_idx)


def kernel(e1_idx, rel_idx, e2_idx, emb_ent, emb_rel):
    return _transe(
        e1_idx.astype(jnp.int32),
        rel_idx.astype(jnp.int32),
        e2_idx.astype(jnp.int32),
        emb_ent.astype(jnp.float32),
        emb_rel.astype(jnp.float32),
    )


# speedup vs baseline: 1.0228x; 1.0228x over previous
import functools

import jax
import jax.numpy as jnp
from jax import lax
from jax.experimental import pallas as pl
from jax.experimental.pallas import tpu as pltpu
from jax.experimental.pallas import tpu_sc as plsc

B = 16384
NE = 1000000
D = 32
L = 16
NW = 32

CHUNKS = 7813
TAIL_CHUNK = CHUNKS - 1
TAIL_BASE = TAIL_CHUNK * 128
CH_PER = 245
PH_CH = 4
PH_E = PH_CH * 128
NPH = 62
QTOT = 2 * B
GROWS = QTOT + L
QPW = B // NW
QPIECE = 2048


def _rsqrt(x):
    i = plsc.bitcast(x, jnp.int32)
    i = jnp.int32(0x5F3759DF) - (i >> 1)
    y = plsc.bitcast(i, jnp.float32)
    for _ in range(3):
        y = y * (1.5 - 0.5 * x * y * y)
    return y


def _gather_body(entT_hbm, tail_hbm, e1_hbm, e2_hbm, g_hbm,
                 epiece, mpk, mpk2, qb2, rowstage, slab3,
                 starts_s, curs_s, dsem0, dsem1,
                 ssem0, ssem1, ssem2, ssem3, ssem4, ssem5, ssem6, ssem7):
    c = lax.axis_index("c")
    s = lax.axis_index("s")
    wid = s * 2 + c
    lo_ch = wid * CH_PER
    hi_ch = jnp.minimum(lo_ch + CH_PER, CHUNKS)
    base_e = lo_ch * 128
    span_e = jnp.minimum(hi_ch * 128, NE) - base_e

    iota = lax.iota(jnp.int32, L)

    def issue(ph, bsel, sem):
        for k in range(PH_CH):
            cg = lo_ch + ph * PH_CH + k
            valid = cg < hi_ch
            is_tail = cg == TAIL_CHUNK
            src_off = pl.multiple_of(
                jnp.where(valid & (~is_tail), cg, 0) * 128, 128)
            dst = slab3.at[bsel, :, pl.ds(k * 128, 128)]

            @pl.when(valid & (~is_tail))
            def _():
                pltpu.async_copy(
                    entT_hbm.at[:, pl.ds(src_off, 128)], dst, sem)

            @pl.when(valid & is_tail)
            def _():
                pltpu.async_copy(tail_hbm, dst, sem)

            @pl.when(~valid)
            def _():
                pltpu.async_copy(
                    entT_hbm.at[:, pl.ds(0, 128)], dst, sem)

    issue(0, 0, dsem0)
    issue(1, 1, dsem1)

    pltpu.sync_copy(e1_hbm, mpk2.at[pl.ds(0, B)])
    pltpu.sync_copy(e2_hbm, mpk2.at[pl.ds(B, B)])

    def match(i, n):
        evec = mpk2[pl.ds(i * L, L)]
        qvec = i * L + iota
        eloc = evec - base_e
        mask = (eloc >= 0) & (eloc < span_e)
        packed = eloc * 32768 + qvec
        plsc.store_compressed(mpk.at[pl.ds(n, L)], packed, mask=mask)
        return n + plsc.all_reduce_population_count(mask)[0]

    m_total = lax.fori_loop(0, QTOT // L, match, 0)

    for ph in range(NPH + 1):
        curs_s[ph] = 0

    def hist(i, _):
        pk = mpk[pl.ds(i * L, L)]
        valid = (i * L + iota) < m_total
        pvec = jnp.where(valid, pk >> 24, NPH)
        for j in range(L):
            pj = pvec[j]
            curs_s[pj] = curs_s[pj] + 1
        return 0

    lax.fori_loop(0, (m_total + L - 1) // L, hist, 0)

    starts_s[0] = 0
    for ph in range(NPH + 1):
        starts_s[ph + 1] = starts_s[ph] + curs_s[ph]
    for ph in range(NPH + 1):
        curs_s[ph] = starts_s[ph]

    lane0 = iota == 0
    zi = jnp.zeros((L,), jnp.int32)

    def place(i, _):
        pk = mpk[pl.ds(i * L, L)]
        valid = (i * L + iota) < m_total
        pvec = jnp.where(valid, pk >> 24, NPH)
        for j in range(L):
            pj = pvec[j]
            pos = curs_s[pj]
            curs_s[pj] = pos + 1
            plsc.store_scatter(mpk2, [zi + pos], zi + pk[j], mask=lane0)
        return 0

    lax.fori_loop(0, (m_total + L - 1) // L, place, 0)

    ssems = [ssem0, ssem1, ssem2, ssem3, ssem4, ssem5, ssem6, ssem7]
    dsems = [dsem0, dsem1]

    def _drain_scatter(sk):
        pltpu.make_async_copy(
            rowstage.at[pl.ds(0, L), :], g_hbm.at[pl.ds(0, L), :], sk).wait()

    def phase(ph, gi):
        bsel = lax.rem(ph, 2)

        for bb in range(2):
            @pl.when(bsel == bb)
            def _(bb=bb):
                pltpu.make_async_copy(
                    entT_hbm.at[:, pl.ds(0, PH_E)], slab3.at[bb],
                    dsems[bb]).wait()

        n0 = starts_s[ph]
        n1 = starts_s[ph + 1]
        slab_lo = ph * PH_E
        b16 = jnp.zeros((L,), jnp.int32) + bsel

        def group(g, gi2):
            slot = gi2 & 7
            i0 = n0 + g * L
            pk = mpk2[pl.ds(i0, L)]
            valid = (i0 + iota) < n1
            eloc = jnp.where(valid, (pk >> 15) - slab_lo, 0)
            qv = jnp.where(valid, pk & 32767, QTOT + iota)
            for sk in range(8):
                @pl.when((slot == sk) & (gi2 >= 8))
                def _(sk=sk):
                    _drain_scatter(ssems[sk])
            rows = slot * L + iota
            for d in range(D):
                dv = jnp.full((L,), d, jnp.int32)
                v = plsc.load_gather(slab3, [b16, dv, eloc])
                col = (16 * (d // 16)) + ((d + qv) & 15)
                plsc.store_scatter(rowstage, [rows, col], v)
            plsc.store_scatter(qb2, [jnp.zeros((L,), jnp.int32) + slot, iota],
                               qv)
            for sk in range(8):
                @pl.when(slot == sk)
                def _(sk=sk):
                    pltpu.async_copy(
                        rowstage.at[pl.ds(sk * L, L), :],
                        g_hbm.at[qb2.at[sk]], ssems[sk])
            return gi2 + 1

        ngroups = (n1 - n0 + L - 1) // L
        gi = lax.fori_loop(0, ngroups, group, gi)

        for bb in range(2):
            @pl.when((bsel == bb) & (ph + 2 < NPH))
            def _(bb=bb):
                issue(ph + 2, bb, dsems[bb])

        return gi

    gi = lax.fori_loop(0, NPH, phase, 0)

    for sk in range(8):
        @pl.when(sk < jnp.minimum(gi, 8))
        def _(sk=sk):
            _drain_scatter(ssems[sk])


def _compute_body(g_hbm, relt_hbm, ridx_hbm, out_hbm,
                  relv, ridx, hbuf, tbuf, outv, dsem):
    c = lax.axis_index("c")
    s = lax.axis_index("s")
    wid = s * 2 + c
    qlo = wid * QPW

    pltpu.sync_copy(relt_hbm, relv)
    pltpu.sync_copy(ridx_hbm.at[pl.ds(qlo, QPW)], ridx)

    iota = lax.iota(jnp.int32, L)
    cols = [((iota + m) & (L - 1)) for m in range(L)]
    zero = jnp.zeros((L,), jnp.float32)

    for p in range(QPW // 128):
        pltpu.sync_copy(g_hbm.at[pl.ds(qlo + p * 128, 128), :], hbuf)
        pltpu.sync_copy(g_hbm.at[pl.ds(B + qlo + p * 128, 128), :], tbuf)

        def group(g, _, p=p):
            rows = g * L + iota
            qbase = qlo + p * 128 + g * L
            qb15 = qbase & 15
            rho = ridx[pl.ds(p * 128 + g * L, L)]
            hh = tt = rr = hr = ht = rt = zero
            for m in range(L):
                for hi in (0, L):
                    colv = cols[m] + hi if hi else cols[m]
                    f = hi + ((m - qb15) & 15)
                    fv = jnp.zeros((L,), jnp.int32) + f
                    h = plsc.load_gather(hbuf, [rows, colv])
                    t = plsc.load_gather(tbuf, [rows, colv])
                    r = plsc.load_gather(relv, [fv, rho])
                    hh = hh + h * h
                    tt = tt + t * t
                    rr = rr + r * r
                    hr = hr + h * r
                    ht = ht + h * t
                    rt = rt + r * t
            ch = _rsqrt(jnp.maximum(hh, 1e-24))
            ct = _rsqrt(jnp.maximum(tt, 1e-24))
            eh = hh * ch * ch
            et = tt * ct * ct
            d2 = eh + et + rr + 2.0 * (hr * ch - ht * (ch * ct) - rt * ct)
            d2 = jnp.maximum(d2, 0.0)
            dist = d2 * _rsqrt(jnp.maximum(d2, 1e-30))
            plsc.store_scatter(outv, [p * 128 + rows], dist)
            return 0

        lax.fori_loop(0, 128 // L, group, 0)

    pltpu.sync_copy(outv, out_hbm.at[pl.ds(qlo, QPW)])


@jax.jit
def _transe(e1_idx, rel_idx, e2_idx, emb_ent, emb_rel):
    entT = emb_ent.T
    tailT = jnp.pad(emb_ent[TAIL_BASE:].T, ((0, 0), (0, 64)))
    relT = jnp.pad(emb_rel.T, ((0, 0), (0, 24)))
    mesh = plsc.VectorSubcoreMesh(core_axis_name="c", subcore_axis_name="s")
    cp = pltpu.CompilerParams(
        needs_layout_passes=False, use_tc_tiling_on_sc=True)

    gather = pl.kernel(
        _gather_body,
        out_type=jax.ShapeDtypeStruct((GROWS, 128), jnp.float32),
        mesh=mesh,
        compiler_params=cp,
        scratch_types=[
            pltpu.VMEM((QPIECE,), jnp.int32),
            pltpu.VMEM((QTOT + L,), jnp.int32),
            pltpu.VMEM((QTOT + L,), jnp.int32),
            pltpu.VMEM((8, L), jnp.int32),
            pltpu.VMEM((128, 128), jnp.float32),
            pltpu.VMEM((2, D, PH_E), jnp.float32),
            pltpu.SMEM((NPH + 2,), jnp.int32),
            pltpu.SMEM((NPH + 1,), jnp.int32),
        ] + [pltpu.SemaphoreType.DMA] * 10,
    )
    compute = pl.kernel(
        _compute_body,
        out_type=jax.ShapeDtypeStruct((B,), jnp.float32),
        mesh=mesh,
        compiler_params=cp,
        scratch_types=[
            pltpu.VMEM((D, 1024), jnp.float32),
            pltpu.VMEM((QPW,), jnp.int32),
            pltpu.VMEM((128, 128), jnp.float32),
            pltpu.VMEM((128, 128), jnp.float32),
            pltpu.VMEM((QPW,), jnp.float32),
            pltpu.SemaphoreType.DMA,
        ],
    )
    g = gather(entT, tailT, e1_idx, e2_idx)
    return compute(g, relT, rel_idx)


def kernel(e1_idx, rel_idx, e2_idx, emb_ent, emb_rel):
    return _transe(
        e1_idx.astype(jnp.int32),
        rel_idx.astype(jnp.int32),
        e2_idx.astype(jnp.int32),
        emb_ent.astype(jnp.float32),
        emb_rel.astype(jnp.float32),
    )
